# baseline (speedup 1.0000x reference)
import functools

import jax
import jax.numpy as jnp
from jax import lax
from jax.experimental import pallas as pl
from jax.experimental.pallas import tpu as pltpu
import jax.experimental.pallas.tpu_sc as plsc

N = 10000
E = 320000
D = 128
NC = 2
NS = 16
NW = NC * NS
CH = 80
NBUF = 10
CPT = 130
EPT = CPT * CH
EPAD = NW * EPT
TRASH = 128
RPT = 624
RREM = N - NS * RPT
WIN = 8

_sc_mesh = plsc.VectorSubcoreMesh(core_axis_name="c", subcore_axis_name="s")


def _deg_body(dst_hbm, ones_hbm, hist_hbm, dst_vm, ones_v, ssem, acc):
    cid = lax.axis_index("c")
    sid = lax.axis_index("s")
    wid = sid * NC + cid
    r0 = pl.multiple_of(sid * RPT, 8)
    pltpu.sync_copy(ones_hbm.at[pl.ds(r0, RPT)], acc.at[pl.ds(r0, RPT)])

    @pl.when(sid == 0)
    def _():
        pltpu.sync_copy(ones_hbm.at[pl.ds(NS * RPT, RREM)],
                        acc.at[pl.ds(NS * RPT, RREM)])

    pltpu.sync_copy(dst_hbm.at[wid], dst_vm)
    pltpu.sync_copy(ones_hbm.at[pl.ds(0, CH)], ones_v)
    plsc.subcore_barrier()

    def body(k, carry):
        @pl.when(k >= WIN)
        def _():
            pltpu.make_async_copy(ones_v, acc.at[dst_vm.at[0]], ssem).wait()

        pltpu.make_async_copy(ones_v, acc.at[dst_vm.at[k]], ssem).start(add=True)
        return carry

    lax.fori_loop(0, CPT, body, 0)
    for _ in range(WIN):
        pltpu.make_async_copy(ones_v, acc.at[dst_vm.at[0]], ssem).wait()
    plsc.subcore_barrier()
    pltpu.sync_copy(acc.at[pl.ds(r0, RPT)], hist_hbm.at[cid].at[pl.ds(r0, RPT)])

    @pl.when(sid == 0)
    def _():
        pltpu.sync_copy(acc.at[pl.ds(NS * RPT, RREM)],
                        hist_hbm.at[cid].at[pl.ds(NS * RPT, RREM)])


_deg_kernel = functools.partial(
    pl.kernel,
    out_type=jax.ShapeDtypeStruct((NC, N, 16), jnp.float32),
    mesh=_sc_mesh,
    scratch_types=[
        pltpu.VMEM((CPT, CH), jnp.int32),
        pltpu.VMEM((CH, 16), jnp.float32),
        pltpu.SemaphoreType.DMA,
        pltpu.VMEM_SHARED((N + TRASH, 16), jnp.float32),
    ],
)(_deg_body)


DH = D // 2


def _agg_body(shl_hbm, shr_hbm, src_hbm, dst_hbm, out_hbm,
              src_vm, dst_vm, rows_v, gsem, ssem, acc):
    cid = lax.axis_index("c")
    sid = lax.axis_index("s")
    wid = sid * NC + cid
    r0 = pl.multiple_of(sid * RPT, 8)

    e0 = pl.multiple_of(wid * EPT, 8)
    pltpu.sync_copy(src_hbm.at[pl.ds(e0, EPT)], src_vm)
    pltpu.sync_copy(dst_hbm.at[wid], dst_vm)

    for p, sh_hbm in ((0, shl_hbm), (1, shr_hbm)):
        pltpu.sync_copy(sh_hbm.at[pl.ds(r0, RPT)], acc.at[pl.ds(r0, RPT)])

        @pl.when(sid == 0)
        def _():
            pltpu.sync_copy(sh_hbm.at[pl.ds(NS * RPT, RREM)],
                            acc.at[pl.ds(NS * RPT, RREM)])

        plsc.subcore_barrier()

        def _gather(k, b):
            off = pl.multiple_of(k * CH, 8)
            pltpu.make_async_copy(sh_hbm.at[src_vm.at[pl.ds(off, CH)]],
                                  rows_v.at[b], gsem.at[b]).start()

        def _wait_gather(b):
            pltpu.make_async_copy(sh_hbm.at[src_vm.at[pl.ds(0, CH)]],
                                  rows_v.at[b], gsem.at[b]).wait()

        def _scatter(k, b):
            pltpu.make_async_copy(rows_v.at[b], acc.at[dst_vm.at[k]],
                                  ssem.at[b]).start(add=True)

        def _wait_scatter(b):
            pltpu.make_async_copy(rows_v.at[b], acc.at[dst_vm.at[0]],
                                  ssem.at[b]).wait()

        for b in range(NBUF - 1):
            _gather(b, b)

        def outer(t, carry):
            k0 = NBUF * t
            for b in range(NBUF):
                k = k0 + b
                _wait_gather(b)
                _scatter(k, b)
                bn = (b + NBUF - 1) % NBUF
                if b == 0:
                    @pl.when(t >= 1)
                    def _():
                        _wait_scatter(bn)
                        _gather(k + NBUF - 1, bn)

                    @pl.when(t < 1)
                    def _():
                        _gather(k + NBUF - 1, bn)
                else:
                    @pl.when(k + NBUF - 1 < CPT)
                    def _():
                        _wait_scatter(bn)
                        _gather(k + NBUF - 1, bn)
            return carry

        lax.fori_loop(0, CPT // NBUF, outer, 0)
        for b in range(NBUF):
            _wait_scatter(b)
        plsc.subcore_barrier()
        pltpu.sync_copy(acc.at[pl.ds(r0, RPT)],
                        out_hbm.at[cid].at[p].at[pl.ds(r0, RPT)])

        @pl.when(sid == 0)
        def _():
            pltpu.sync_copy(acc.at[pl.ds(NS * RPT, RREM)],
                            out_hbm.at[cid].at[p].at[pl.ds(NS * RPT, RREM)])


_agg_kernel = functools.partial(
    pl.kernel,
    out_type=jax.ShapeDtypeStruct((NC, 2, N, DH), jnp.float32),
    mesh=_sc_mesh,
    compiler_params=pltpu.CompilerParams(use_tc_tiling_on_sc=False),
    scratch_types=[
        pltpu.VMEM((EPT,), jnp.int32),
        pltpu.VMEM((CPT, CH), jnp.int32),
        pltpu.VMEM((NBUF, CH, DH), jnp.float32),
        pltpu.SemaphoreType.DMA((NBUF,)),
        pltpu.SemaphoreType.DMA((NBUF,)),
        pltpu.VMEM_SHARED((N + TRASH, DH), jnp.float32),
    ],
)(_agg_body)


_RB = 1000


def _mm_body(x_ref, w_ref, hist_ref, shl_ref, shr_ref):
    deg = hist_ref[0, :, 0] + hist_ref[1, :, 0] - 1.0
    u = lax.rsqrt(deg)
    h = jnp.dot(x_ref[...], w_ref[...], preferred_element_type=jnp.float32)
    sh = h * u[:, None]
    shl_ref[...] = sh[:, :DH]
    shr_ref[...] = sh[:, DH:]


def _mm_call(x, w, hist):
    return pl.pallas_call(
        _mm_body,
        grid=(N // _RB,),
        in_specs=[
            pl.BlockSpec((_RB, D), lambda i: (i, 0)),
            pl.BlockSpec((D, D), lambda i: (0, 0)),
            pl.BlockSpec((NC, _RB, 16), lambda i: (0, i, 0)),
        ],
        out_specs=[
            pl.BlockSpec((_RB, DH), lambda i: (i, 0)),
            pl.BlockSpec((_RB, DH), lambda i: (i, 0)),
        ],
        out_shape=[
            jax.ShapeDtypeStruct((N, DH), jnp.float32),
            jax.ShapeDtypeStruct((N, DH), jnp.float32),
        ],
    )(x, w, hist)


def _fin_body(part_ref, shl_ref, shr_ref, hist_ref, b_ref, o_ref):
    deg = hist_ref[0, :, 0] + hist_ref[1, :, 0] - 1.0
    u = lax.rsqrt(deg)
    aggl = part_ref[0, 0] + part_ref[1, 0] - shl_ref[...]
    aggr = part_ref[0, 1] + part_ref[1, 1] - shr_ref[...]
    agg = jnp.concatenate([aggl, aggr], axis=-1)
    o_ref[...] = jnp.maximum(agg * u[:, None] + b_ref[...][None, :], 0.0)


def _fin_call(part, shl, shr, hist, b):
    return pl.pallas_call(
        _fin_body,
        grid=(N // _RB,),
        in_specs=[
            pl.BlockSpec((NC, 2, _RB, DH), lambda i: (0, 0, i, 0)),
            pl.BlockSpec((_RB, DH), lambda i: (i, 0)),
            pl.BlockSpec((_RB, DH), lambda i: (i, 0)),
            pl.BlockSpec((NC, _RB, 16), lambda i: (0, i, 0)),
            pl.BlockSpec((D,), lambda i: (0,)),
        ],
        out_specs=pl.BlockSpec((_RB, D), lambda i: (i, 0)),
        out_shape=jax.ShapeDtypeStruct((N, D), jnp.float32),
    )(part, shl, shr, hist, b)


def kernel(x, edge_index, W, b):
    npad = EPAD - E
    src = jnp.concatenate(
        [edge_index[0], jnp.zeros((npad,), dtype=jnp.int32)])
    trash = N + (jnp.arange(npad, dtype=jnp.int32) % TRASH)
    dst = jnp.concatenate([edge_index[1], trash]).reshape(NW, CPT, CH)
    ones = jnp.ones((N, 16), dtype=jnp.float32)
    hist = _deg_kernel(dst, ones)
    shl, shr = _mm_call(x, W, hist)
    part = _agg_kernel(shl, shr, src, dst)
    return _fin_call(part, shl, shr, hist, b)


# speedup vs baseline: 1.0150x; 1.0150x over previous
import functools

import jax
import jax.numpy as jnp
from jax import lax
from jax.experimental import pallas as pl
from jax.experimental.pallas import tpu as pltpu
import jax.experimental.pallas.tpu_sc as plsc

N = 10000
E = 320000
D = 128
NC = 2
NS = 16
NW = NC * NS
CH = 80
NBUF = 5
CPT = 130
EPT = CPT * CH
EPAD = NW * EPT
TRASH = 128
RPT = 624
RREM = N - NS * RPT
WIN = 8

_sc_mesh = plsc.VectorSubcoreMesh(core_axis_name="c", subcore_axis_name="s")


def _deg_body(dst_hbm, ones_hbm, hist_hbm, dst_vm, ones_v, ssem, acc):
    cid = lax.axis_index("c")
    sid = lax.axis_index("s")
    wid = sid * NC + cid
    r0 = pl.multiple_of(sid * RPT, 8)
    pltpu.sync_copy(ones_hbm.at[pl.ds(r0, RPT)], acc.at[pl.ds(r0, RPT)])

    @pl.when(sid == 0)
    def _():
        pltpu.sync_copy(ones_hbm.at[pl.ds(NS * RPT, RREM)],
                        acc.at[pl.ds(NS * RPT, RREM)])

    pltpu.sync_copy(dst_hbm.at[wid], dst_vm)
    pltpu.sync_copy(ones_hbm.at[pl.ds(0, CH)], ones_v)
    plsc.subcore_barrier()

    def body(k, carry):
        @pl.when(k >= WIN)
        def _():
            pltpu.make_async_copy(ones_v, acc.at[dst_vm.at[0]], ssem).wait()

        pltpu.make_async_copy(ones_v, acc.at[dst_vm.at[k]], ssem).start(add=True)
        return carry

    lax.fori_loop(0, CPT, body, 0)
    for _ in range(WIN):
        pltpu.make_async_copy(ones_v, acc.at[dst_vm.at[0]], ssem).wait()
    plsc.subcore_barrier()
    pltpu.sync_copy(acc.at[pl.ds(r0, RPT)], hist_hbm.at[cid].at[pl.ds(r0, RPT)])

    @pl.when(sid == 0)
    def _():
        pltpu.sync_copy(acc.at[pl.ds(NS * RPT, RREM)],
                        hist_hbm.at[cid].at[pl.ds(NS * RPT, RREM)])


_deg_kernel = functools.partial(
    pl.kernel,
    out_type=jax.ShapeDtypeStruct((NC, N, 16), jnp.float32),
    mesh=_sc_mesh,
    scratch_types=[
        pltpu.VMEM((CPT, CH), jnp.int32),
        pltpu.VMEM((CH, 16), jnp.float32),
        pltpu.SemaphoreType.DMA,
        pltpu.VMEM_SHARED((N + TRASH, 16), jnp.float32),
    ],
)(_deg_body)


DH = D // 2


def _agg_body(shl_hbm, shr_hbm, src_hbm, dst_hbm, out_hbm,
              src_vm, dst_vm, rows_v, gsem, ssem, acc):
    cid = lax.axis_index("c")
    sid = lax.axis_index("s")
    wid = sid * NC + cid
    r0 = pl.multiple_of(sid * RPT, 8)

    e0 = pl.multiple_of(wid * EPT, 8)
    pltpu.sync_copy(src_hbm.at[pl.ds(e0, EPT)], src_vm)
    pltpu.sync_copy(dst_hbm.at[wid], dst_vm)

    for p, sh_hbm in ((0, shl_hbm), (1, shr_hbm)):
        pltpu.sync_copy(sh_hbm.at[pl.ds(r0, RPT)], acc.at[pl.ds(r0, RPT)])

        @pl.when(sid == 0)
        def _():
            pltpu.sync_copy(sh_hbm.at[pl.ds(NS * RPT, RREM)],
                            acc.at[pl.ds(NS * RPT, RREM)])

        plsc.subcore_barrier()

        def _gather(k, b):
            off = pl.multiple_of(k * CH, 8)
            pltpu.make_async_copy(sh_hbm.at[src_vm.at[pl.ds(off, CH)]],
                                  rows_v.at[b], gsem.at[b]).start()

        def _wait_gather(b):
            pltpu.make_async_copy(sh_hbm.at[src_vm.at[pl.ds(0, CH)]],
                                  rows_v.at[b], gsem.at[b]).wait()

        def _scatter(k, b):
            pltpu.make_async_copy(rows_v.at[b], acc.at[dst_vm.at[k]],
                                  ssem.at[b]).start(add=True)

        def _wait_scatter(b):
            pltpu.make_async_copy(rows_v.at[b], acc.at[dst_vm.at[0]],
                                  ssem.at[b]).wait()

        for b in range(NBUF - 1):
            _gather(b, b)

        def outer(t, carry):
            k0 = NBUF * t
            for b in range(NBUF):
                k = k0 + b
                _wait_gather(b)
                _scatter(k, b)
                bn = (b + NBUF - 1) % NBUF
                if b == 0:
                    @pl.when(t >= 1)
                    def _():
                        _wait_scatter(bn)
                        _gather(k + NBUF - 1, bn)

                    @pl.when(t < 1)
                    def _():
                        _gather(k + NBUF - 1, bn)
                else:
                    @pl.when(k + NBUF - 1 < CPT)
                    def _():
                        _wait_scatter(bn)
                        _gather(k + NBUF - 1, bn)
            return carry

        lax.fori_loop(0, CPT // NBUF, outer, 0)
        for b in range(NBUF):
            _wait_scatter(b)
        plsc.subcore_barrier()
        pltpu.sync_copy(acc.at[pl.ds(r0, RPT)],
                        out_hbm.at[cid].at[p].at[pl.ds(r0, RPT)])

        @pl.when(sid == 0)
        def _():
            pltpu.sync_copy(acc.at[pl.ds(NS * RPT, RREM)],
                            out_hbm.at[cid].at[p].at[pl.ds(NS * RPT, RREM)])


_agg_kernel = functools.partial(
    pl.kernel,
    out_type=jax.ShapeDtypeStruct((NC, 2, N, DH), jnp.float32),
    mesh=_sc_mesh,
    compiler_params=pltpu.CompilerParams(use_tc_tiling_on_sc=False),
    scratch_types=[
        pltpu.VMEM((EPT,), jnp.int32),
        pltpu.VMEM((CPT, CH), jnp.int32),
        pltpu.VMEM((NBUF, CH, DH), jnp.float32),
        pltpu.SemaphoreType.DMA((NBUF,)),
        pltpu.SemaphoreType.DMA((NBUF,)),
        pltpu.VMEM_SHARED((N + TRASH, DH), jnp.float32),
    ],
)(_agg_body)


_RB = 1000


def _mm_body(x_ref, w_ref, hist_ref, shl_ref, shr_ref):
    deg = hist_ref[0, :, 0] + hist_ref[1, :, 0] - 1.0
    u = lax.rsqrt(deg)
    h = jnp.dot(x_ref[...], w_ref[...], preferred_element_type=jnp.float32)
    sh = h * u[:, None]
    shl_ref[...] = sh[:, :DH]
    shr_ref[...] = sh[:, DH:]


def _mm_call(x, w, hist):
    return pl.pallas_call(
        _mm_body,
        grid=(N // _RB,),
        in_specs=[
            pl.BlockSpec((_RB, D), lambda i: (i, 0)),
            pl.BlockSpec((D, D), lambda i: (0, 0)),
            pl.BlockSpec((NC, _RB, 16), lambda i: (0, i, 0)),
        ],
        out_specs=[
            pl.BlockSpec((_RB, DH), lambda i: (i, 0)),
            pl.BlockSpec((_RB, DH), lambda i: (i, 0)),
        ],
        out_shape=[
            jax.ShapeDtypeStruct((N, DH), jnp.float32),
            jax.ShapeDtypeStruct((N, DH), jnp.float32),
        ],
    )(x, w, hist)


def _fin_body(part_ref, shl_ref, shr_ref, hist_ref, b_ref, o_ref):
    deg = hist_ref[0, :, 0] + hist_ref[1, :, 0] - 1.0
    u = lax.rsqrt(deg)
    aggl = part_ref[0, 0] + part_ref[1, 0] - shl_ref[...]
    aggr = part_ref[0, 1] + part_ref[1, 1] - shr_ref[...]
    agg = jnp.concatenate([aggl, aggr], axis=-1)
    o_ref[...] = jnp.maximum(agg * u[:, None] + b_ref[...][None, :], 0.0)


def _fin_call(part, shl, shr, hist, b):
    return pl.pallas_call(
        _fin_body,
        grid=(N // _RB,),
        in_specs=[
            pl.BlockSpec((NC, 2, _RB, DH), lambda i: (0, 0, i, 0)),
            pl.BlockSpec((_RB, DH), lambda i: (i, 0)),
            pl.BlockSpec((_RB, DH), lambda i: (i, 0)),
            pl.BlockSpec((NC, _RB, 16), lambda i: (0, i, 0)),
            pl.BlockSpec((D,), lambda i: (0,)),
        ],
        out_specs=pl.BlockSpec((_RB, D), lambda i: (i, 0)),
        out_shape=jax.ShapeDtypeStruct((N, D), jnp.float32),
    )(part, shl, shr, hist, b)


def kernel(x, edge_index, W, b):
    npad = EPAD - E
    src = jnp.concatenate(
        [edge_index[0], jnp.zeros((npad,), dtype=jnp.int32)])
    trash = N + (jnp.arange(npad, dtype=jnp.int32) % TRASH)
    dst = jnp.concatenate([edge_index[1], trash]).reshape(NW, CPT, CH)
    ones = jnp.ones((N, 16), dtype=jnp.float32)
    hist = _deg_kernel(dst, ones)
    shl, shr = _mm_call(x, W, hist)
    part = _agg_kernel(shl, shr, src, dst)
    return _fin_call(part, shl, shr, hist, b)


# speedup vs baseline: 3.4338x; 3.3832x over previous
import functools

import jax
import jax.numpy as jnp
from jax import lax
from jax.experimental import pallas as pl
from jax.experimental.pallas import tpu as pltpu
import jax.experimental.pallas.tpu_sc as plsc

N = 10000
E = 320000
D = 128
NC = 2
NS = 16
NW = NC * NS
CH = 80
NBUF = 5
CPT = 130
EPT = CPT * CH
EPAD = NW * EPT
TRASH = 128
RPT = 624
RREM = N - NS * RPT
WIN = 8

_sc_mesh = plsc.VectorSubcoreMesh(core_axis_name="c", subcore_axis_name="s")


def _deg_body(dst_hbm, ones_hbm, hist_hbm, dst_vm, ones_v, ssem, acc):
    cid = lax.axis_index("c")
    sid = lax.axis_index("s")
    wid = sid * NC + cid
    r0 = pl.multiple_of(sid * RPT, 8)
    pltpu.sync_copy(ones_hbm.at[pl.ds(r0, RPT)], acc.at[pl.ds(r0, RPT)])

    @pl.when(sid == 0)
    def _():
        pltpu.sync_copy(ones_hbm.at[pl.ds(NS * RPT, RREM)],
                        acc.at[pl.ds(NS * RPT, RREM)])

    pltpu.sync_copy(dst_hbm.at[wid], dst_vm)
    pltpu.sync_copy(ones_hbm.at[pl.ds(0, CH)], ones_v)
    plsc.subcore_barrier()

    def body(k, carry):
        @pl.when(k >= WIN)
        def _():
            pltpu.make_async_copy(ones_v, acc.at[dst_vm.at[0]], ssem).wait()

        pltpu.make_async_copy(ones_v, acc.at[dst_vm.at[k]], ssem).start(add=True)
        return carry

    lax.fori_loop(0, CPT, body, 0)
    for _ in range(WIN):
        pltpu.make_async_copy(ones_v, acc.at[dst_vm.at[0]], ssem).wait()
    plsc.subcore_barrier()
    pltpu.sync_copy(acc.at[pl.ds(r0, RPT)], hist_hbm.at[cid].at[pl.ds(r0, RPT)])

    @pl.when(sid == 0)
    def _():
        pltpu.sync_copy(acc.at[pl.ds(NS * RPT, RREM)],
                        hist_hbm.at[cid].at[pl.ds(NS * RPT, RREM)])


_deg_kernel = functools.partial(
    pl.kernel,
    out_type=jax.ShapeDtypeStruct((NC, N, 16), jnp.float32),
    mesh=_sc_mesh,
    scratch_types=[
        pltpu.VMEM((CPT, CH), jnp.int32),
        pltpu.VMEM((CH, 16), jnp.float32),
        pltpu.SemaphoreType.DMA,
        pltpu.VMEM_SHARED((N + TRASH, 16), jnp.float32),
    ],
)(_deg_body)


DH = D // 2


def _agg_body(shl_hbm, shr_hbm, src_hbm, dst_hbm, out_hbm,
              src_vm, dst_vm, rows_v, gsem, ssem, acc):
    cid = lax.axis_index("c")
    sid = lax.axis_index("s")
    wid = sid * NC + cid
    r0 = pl.multiple_of(sid * RPT, 8)

    e0 = pl.multiple_of(wid * EPT, 8)
    pltpu.sync_copy(src_hbm.at[pl.ds(e0, EPT)], src_vm)
    pltpu.sync_copy(dst_hbm.at[wid], dst_vm)

    for p, sh_hbm in ((0, shl_hbm), (1, shr_hbm)):
        pltpu.sync_copy(sh_hbm.at[pl.ds(r0, RPT)], acc.at[pl.ds(r0, RPT)])

        @pl.when(sid == 0)
        def _():
            pltpu.sync_copy(sh_hbm.at[pl.ds(NS * RPT, RREM)],
                            acc.at[pl.ds(NS * RPT, RREM)])

        plsc.subcore_barrier()

        def _gather(k, b):
            off = pl.multiple_of(k * CH, 8)
            pltpu.make_async_copy(sh_hbm.at[src_vm.at[pl.ds(off, CH)]],
                                  rows_v.at[b], gsem.at[b]).start()

        def _wait_gather(b):
            pltpu.make_async_copy(sh_hbm.at[src_vm.at[pl.ds(0, CH)]],
                                  rows_v.at[b], gsem.at[b]).wait()

        def _scatter(k, b):
            pltpu.make_async_copy(rows_v.at[b], acc.at[dst_vm.at[k]],
                                  ssem.at[b]).start(add=True)

        def _wait_scatter(b):
            pltpu.make_async_copy(rows_v.at[b], acc.at[dst_vm.at[0]],
                                  ssem.at[b]).wait()

        for b in range(NBUF - 1):
            _gather(b, b)

        def outer(t, carry):
            k0 = NBUF * t
            for b in range(NBUF):
                k = k0 + b
                _wait_gather(b)
                _scatter(k, b)
                bn = (b + NBUF - 1) % NBUF
                if b == 0:
                    @pl.when(t >= 1)
                    def _():
                        _wait_scatter(bn)
                        _gather(k + NBUF - 1, bn)

                    @pl.when(t < 1)
                    def _():
                        _gather(k + NBUF - 1, bn)
                else:
                    @pl.when(k + NBUF - 1 < CPT)
                    def _():
                        _wait_scatter(bn)
                        _gather(k + NBUF - 1, bn)
            return carry

        lax.fori_loop(0, CPT // NBUF, outer, 0)
        for b in range(NBUF):
            _wait_scatter(b)
        plsc.subcore_barrier()
        pltpu.sync_copy(acc.at[pl.ds(r0, RPT)],
                        out_hbm.at[cid].at[p].at[pl.ds(r0, RPT)])

        @pl.when(sid == 0)
        def _():
            pltpu.sync_copy(acc.at[pl.ds(NS * RPT, RREM)],
                            out_hbm.at[cid].at[p].at[pl.ds(NS * RPT, RREM)])


_agg_kernel = functools.partial(
    pl.kernel,
    out_type=jax.ShapeDtypeStruct((NC, 2, N, DH), jnp.float32),
    mesh=_sc_mesh,
    compiler_params=pltpu.CompilerParams(use_tc_tiling_on_sc=False),
    scratch_types=[
        pltpu.VMEM((EPT,), jnp.int32),
        pltpu.VMEM((CPT, CH), jnp.int32),
        pltpu.VMEM((NBUF, CH, DH), jnp.float32),
        pltpu.SemaphoreType.DMA((NBUF,)),
        pltpu.SemaphoreType.DMA((NBUF,)),
        pltpu.VMEM_SHARED((N + TRASH, DH), jnp.float32),
    ],
)(_agg_body)


_RB = 1000


def _mm_body(x_ref, w_ref, hist_ref, shl_ref, shr_ref):
    deg = hist_ref[0, :, 0] + hist_ref[1, :, 0] - 1.0
    u = lax.rsqrt(deg)
    h = jnp.dot(x_ref[...], w_ref[...], preferred_element_type=jnp.float32)
    sh = h * u[:, None]
    shl_ref[...] = sh[:, :DH]
    shr_ref[...] = sh[:, DH:]


def _mm_call(x, w, hist):
    return pl.pallas_call(
        _mm_body,
        grid=(N // _RB,),
        in_specs=[
            pl.BlockSpec((_RB, D), lambda i: (i, 0)),
            pl.BlockSpec((D, D), lambda i: (0, 0)),
            pl.BlockSpec((NC, _RB, 16), lambda i: (0, i, 0)),
        ],
        out_specs=[
            pl.BlockSpec((_RB, DH), lambda i: (i, 0)),
            pl.BlockSpec((_RB, DH), lambda i: (i, 0)),
        ],
        out_shape=[
            jax.ShapeDtypeStruct((N, DH), jnp.float32),
            jax.ShapeDtypeStruct((N, DH), jnp.float32),
        ],
    )(x, w, hist)


def _fin_body(part_ref, shl_ref, shr_ref, hist_ref, b_ref, o_ref):
    deg = hist_ref[0, :, 0] + hist_ref[1, :, 0] - 1.0
    u = lax.rsqrt(deg)
    aggl = part_ref[0, 0] + part_ref[1, 0] - shl_ref[...]
    aggr = part_ref[0, 1] + part_ref[1, 1] - shr_ref[...]
    agg = jnp.concatenate([aggl, aggr], axis=-1)
    o_ref[...] = jnp.maximum(agg * u[:, None] + b_ref[...][None, :], 0.0)


def _fin_call(part, shl, shr, hist, b):
    return pl.pallas_call(
        _fin_body,
        grid=(N // _RB,),
        in_specs=[
            pl.BlockSpec((NC, 2, _RB, DH), lambda i: (0, 0, i, 0)),
            pl.BlockSpec((_RB, DH), lambda i: (i, 0)),
            pl.BlockSpec((_RB, DH), lambda i: (i, 0)),
            pl.BlockSpec((NC, _RB, 16), lambda i: (0, i, 0)),
            pl.BlockSpec((D,), lambda i: (0,)),
        ],
        out_specs=pl.BlockSpec((_RB, D), lambda i: (i, 0)),
        out_shape=jax.ShapeDtypeStruct((N, D), jnp.float32),
    )(part, shl, shr, hist, b)


def kernel(x, edge_index, W, b):
    npad = EPAD - E
    spread = jnp.arange(npad, dtype=jnp.int32) % N
    src = jnp.concatenate([edge_index[0], spread])
    trash = N + (jnp.arange(npad, dtype=jnp.int32) % TRASH)
    dst = jnp.concatenate([edge_index[1], trash]).reshape(NW, CPT, CH)
    ones = jnp.ones((N, 16), dtype=jnp.float32)
    hist = _deg_kernel(dst, ones)
    shl, shr = _mm_call(x, W, hist)
    part = _agg_kernel(shl, shr, src, dst)
    return _fin_call(part, shl, shr, hist, b)


# speedup vs baseline: 3.4914x; 1.0168x over previous
import functools

import jax
import jax.numpy as jnp
from jax import lax
from jax.experimental import pallas as pl
from jax.experimental.pallas import tpu as pltpu
import jax.experimental.pallas.tpu_sc as plsc

N = 10000
E = 320000
D = 128
NC = 2
NS = 16
NW = NC * NS
CH = 80
NBUF = 10
CPT = 130
EPT = CPT * CH
EPAD = NW * EPT
TRASH = 128
RPT = 624
RREM = N - NS * RPT
WIN = 8

_sc_mesh = plsc.VectorSubcoreMesh(core_axis_name="c", subcore_axis_name="s")


def _deg_body(dst_hbm, ones_hbm, hist_hbm, dst_vm, ones_v, ssem, acc):
    cid = lax.axis_index("c")
    sid = lax.axis_index("s")
    wid = sid * NC + cid
    r0 = pl.multiple_of(sid * RPT, 8)
    pltpu.sync_copy(ones_hbm.at[pl.ds(r0, RPT)], acc.at[pl.ds(r0, RPT)])

    @pl.when(sid == 0)
    def _():
        pltpu.sync_copy(ones_hbm.at[pl.ds(NS * RPT, RREM)],
                        acc.at[pl.ds(NS * RPT, RREM)])

    pltpu.sync_copy(dst_hbm.at[wid], dst_vm)
    pltpu.sync_copy(ones_hbm.at[pl.ds(0, CH)], ones_v)
    plsc.subcore_barrier()

    def body(k, carry):
        @pl.when(k >= WIN)
        def _():
            pltpu.make_async_copy(ones_v, acc.at[dst_vm.at[0]], ssem).wait()

        pltpu.make_async_copy(ones_v, acc.at[dst_vm.at[k]], ssem).start(add=True)
        return carry

    lax.fori_loop(0, CPT, body, 0)
    for _ in range(WIN):
        pltpu.make_async_copy(ones_v, acc.at[dst_vm.at[0]], ssem).wait()
    plsc.subcore_barrier()
    pltpu.sync_copy(acc.at[pl.ds(r0, RPT)], hist_hbm.at[cid].at[pl.ds(r0, RPT)])

    @pl.when(sid == 0)
    def _():
        pltpu.sync_copy(acc.at[pl.ds(NS * RPT, RREM)],
                        hist_hbm.at[cid].at[pl.ds(NS * RPT, RREM)])


_deg_kernel = functools.partial(
    pl.kernel,
    out_type=jax.ShapeDtypeStruct((NC, N, 16), jnp.float32),
    mesh=_sc_mesh,
    scratch_types=[
        pltpu.VMEM((CPT, CH), jnp.int32),
        pltpu.VMEM((CH, 16), jnp.float32),
        pltpu.SemaphoreType.DMA,
        pltpu.VMEM_SHARED((N + TRASH, 16), jnp.float32),
    ],
)(_deg_body)


DH = D // 2


def _agg_body(shl_hbm, shr_hbm, src_hbm, dst_hbm, out_hbm,
              src_vm, dst_vm, rows_v, gsem, ssem, acc):
    cid = lax.axis_index("c")
    sid = lax.axis_index("s")
    wid = sid * NC + cid
    r0 = pl.multiple_of(sid * RPT, 8)

    e0 = pl.multiple_of(wid * EPT, 8)
    pltpu.sync_copy(src_hbm.at[pl.ds(e0, EPT)], src_vm)
    pltpu.sync_copy(dst_hbm.at[wid], dst_vm)

    for p, sh_hbm in ((0, shl_hbm), (1, shr_hbm)):
        pltpu.sync_copy(sh_hbm.at[pl.ds(r0, RPT)], acc.at[pl.ds(r0, RPT)])

        @pl.when(sid == 0)
        def _():
            pltpu.sync_copy(sh_hbm.at[pl.ds(NS * RPT, RREM)],
                            acc.at[pl.ds(NS * RPT, RREM)])

        plsc.subcore_barrier()

        def _gather(k, b):
            off = pl.multiple_of(k * CH, 8)
            pltpu.make_async_copy(sh_hbm.at[src_vm.at[pl.ds(off, CH)]],
                                  rows_v.at[b], gsem.at[b]).start()

        def _wait_gather(b):
            pltpu.make_async_copy(sh_hbm.at[src_vm.at[pl.ds(0, CH)]],
                                  rows_v.at[b], gsem.at[b]).wait()

        def _scatter(k, b):
            pltpu.make_async_copy(rows_v.at[b], acc.at[dst_vm.at[k]],
                                  ssem.at[b]).start(add=True)

        def _wait_scatter(b):
            pltpu.make_async_copy(rows_v.at[b], acc.at[dst_vm.at[0]],
                                  ssem.at[b]).wait()

        for b in range(NBUF - 1):
            _gather(b, b)

        def outer(t, carry):
            k0 = NBUF * t
            for b in range(NBUF):
                k = k0 + b
                _wait_gather(b)
                _scatter(k, b)
                bn = (b + NBUF - 1) % NBUF
                if b == 0:
                    @pl.when(t >= 1)
                    def _():
                        _wait_scatter(bn)
                        _gather(k + NBUF - 1, bn)

                    @pl.when(t < 1)
                    def _():
                        _gather(k + NBUF - 1, bn)
                else:
                    @pl.when(k + NBUF - 1 < CPT)
                    def _():
                        _wait_scatter(bn)
                        _gather(k + NBUF - 1, bn)
            return carry

        lax.fori_loop(0, CPT // NBUF, outer, 0)
        for b in range(NBUF):
            _wait_scatter(b)
        plsc.subcore_barrier()
        pltpu.sync_copy(acc.at[pl.ds(r0, RPT)],
                        out_hbm.at[cid].at[p].at[pl.ds(r0, RPT)])

        @pl.when(sid == 0)
        def _():
            pltpu.sync_copy(acc.at[pl.ds(NS * RPT, RREM)],
                            out_hbm.at[cid].at[p].at[pl.ds(NS * RPT, RREM)])


_agg_kernel = functools.partial(
    pl.kernel,
    out_type=jax.ShapeDtypeStruct((NC, 2, N, DH), jnp.float32),
    mesh=_sc_mesh,
    compiler_params=pltpu.CompilerParams(use_tc_tiling_on_sc=False),
    scratch_types=[
        pltpu.VMEM((EPT,), jnp.int32),
        pltpu.VMEM((CPT, CH), jnp.int32),
        pltpu.VMEM((NBUF, CH, DH), jnp.float32),
        pltpu.SemaphoreType.DMA((NBUF,)),
        pltpu.SemaphoreType.DMA((NBUF,)),
        pltpu.VMEM_SHARED((N + TRASH, DH), jnp.float32),
    ],
)(_agg_body)


_RB = 1000


def _mm_body(x_ref, w_ref, hist_ref, shl_ref, shr_ref):
    deg = hist_ref[0, :, 0] + hist_ref[1, :, 0] - 1.0
    u = lax.rsqrt(deg)
    h = jnp.dot(x_ref[...], w_ref[...], preferred_element_type=jnp.float32)
    sh = h * u[:, None]
    shl_ref[...] = sh[:, :DH]
    shr_ref[...] = sh[:, DH:]


def _mm_call(x, w, hist):
    return pl.pallas_call(
        _mm_body,
        grid=(N // _RB,),
        in_specs=[
            pl.BlockSpec((_RB, D), lambda i: (i, 0)),
            pl.BlockSpec((D, D), lambda i: (0, 0)),
            pl.BlockSpec((NC, _RB, 16), lambda i: (0, i, 0)),
        ],
        out_specs=[
            pl.BlockSpec((_RB, DH), lambda i: (i, 0)),
            pl.BlockSpec((_RB, DH), lambda i: (i, 0)),
        ],
        out_shape=[
            jax.ShapeDtypeStruct((N, DH), jnp.float32),
            jax.ShapeDtypeStruct((N, DH), jnp.float32),
        ],
    )(x, w, hist)


def _fin_body(part_ref, shl_ref, shr_ref, hist_ref, b_ref, o_ref):
    deg = hist_ref[0, :, 0] + hist_ref[1, :, 0] - 1.0
    u = lax.rsqrt(deg)
    aggl = part_ref[0, 0] + part_ref[1, 0] - shl_ref[...]
    aggr = part_ref[0, 1] + part_ref[1, 1] - shr_ref[...]
    agg = jnp.concatenate([aggl, aggr], axis=-1)
    o_ref[...] = jnp.maximum(agg * u[:, None] + b_ref[...][None, :], 0.0)


def _fin_call(part, shl, shr, hist, b):
    return pl.pallas_call(
        _fin_body,
        grid=(N // _RB,),
        in_specs=[
            pl.BlockSpec((NC, 2, _RB, DH), lambda i: (0, 0, i, 0)),
            pl.BlockSpec((_RB, DH), lambda i: (i, 0)),
            pl.BlockSpec((_RB, DH), lambda i: (i, 0)),
            pl.BlockSpec((NC, _RB, 16), lambda i: (0, i, 0)),
            pl.BlockSpec((D,), lambda i: (0,)),
        ],
        out_specs=pl.BlockSpec((_RB, D), lambda i: (i, 0)),
        out_shape=jax.ShapeDtypeStruct((N, D), jnp.float32),
    )(part, shl, shr, hist, b)


def kernel(x, edge_index, W, b):
    npad = EPAD - E
    spread = jnp.arange(npad, dtype=jnp.int32) % N
    src = jnp.concatenate([edge_index[0], spread])
    trash = N + (jnp.arange(npad, dtype=jnp.int32) % TRASH)
    dst = jnp.concatenate([edge_index[1], trash]).reshape(NW, CPT, CH)
    ones = jnp.ones((N, 16), dtype=jnp.float32)
    hist = _deg_kernel(dst, ones)
    shl, shr = _mm_call(x, W, hist)
    part = _agg_kernel(shl, shr, src, dst)
    return _fin_call(part, shl, shr, hist, b)


# speedup vs baseline: 3.5060x; 1.0042x over previous
import functools

import jax
import jax.numpy as jnp
from jax import lax
from jax.experimental import pallas as pl
from jax.experimental.pallas import tpu as pltpu
import jax.experimental.pallas.tpu_sc as plsc

N = 10000
E = 320000
D = 128
NC = 2
NS = 16
NW = NC * NS
CH = 128
NBUF = 5
CPT = 80
EPT = CPT * CH
EPAD = NW * EPT
TRASH = 128
RPT = 624
RREM = N - NS * RPT
WIN = 8

_sc_mesh = plsc.VectorSubcoreMesh(core_axis_name="c", subcore_axis_name="s")


def _deg_body(dst_hbm, ones_hbm, hist_hbm, dst_vm, ones_v, ssem, acc):
    cid = lax.axis_index("c")
    sid = lax.axis_index("s")
    wid = sid * NC + cid
    r0 = pl.multiple_of(sid * RPT, 8)
    pltpu.sync_copy(ones_hbm.at[pl.ds(r0, RPT)], acc.at[pl.ds(r0, RPT)])

    @pl.when(sid == 0)
    def _():
        pltpu.sync_copy(ones_hbm.at[pl.ds(NS * RPT, RREM)],
                        acc.at[pl.ds(NS * RPT, RREM)])

    pltpu.sync_copy(dst_hbm.at[wid], dst_vm)
    pltpu.sync_copy(ones_hbm.at[pl.ds(0, CH)], ones_v)
    plsc.subcore_barrier()

    def body(k, carry):
        @pl.when(k >= WIN)
        def _():
            pltpu.make_async_copy(ones_v, acc.at[dst_vm.at[0]], ssem).wait()

        pltpu.make_async_copy(ones_v, acc.at[dst_vm.at[k]], ssem).start(add=True)
        return carry

    lax.fori_loop(0, CPT, body, 0)
    for _ in range(WIN):
        pltpu.make_async_copy(ones_v, acc.at[dst_vm.at[0]], ssem).wait()
    plsc.subcore_barrier()
    pltpu.sync_copy(acc.at[pl.ds(r0, RPT)], hist_hbm.at[cid].at[pl.ds(r0, RPT)])

    @pl.when(sid == 0)
    def _():
        pltpu.sync_copy(acc.at[pl.ds(NS * RPT, RREM)],
                        hist_hbm.at[cid].at[pl.ds(NS * RPT, RREM)])


_deg_kernel = functools.partial(
    pl.kernel,
    out_type=jax.ShapeDtypeStruct((NC, N, 16), jnp.float32),
    mesh=_sc_mesh,
    scratch_types=[
        pltpu.VMEM((CPT, CH), jnp.int32),
        pltpu.VMEM((CH, 16), jnp.float32),
        pltpu.SemaphoreType.DMA,
        pltpu.VMEM_SHARED((N + TRASH, 16), jnp.float32),
    ],
)(_deg_body)


DH = D // 2


def _agg_body(shl_hbm, shr_hbm, src_hbm, dst_hbm, out_hbm,
              src_vm, dst_vm, rows_v, gsem, ssem, acc):
    cid = lax.axis_index("c")
    sid = lax.axis_index("s")
    wid = sid * NC + cid
    r0 = pl.multiple_of(sid * RPT, 8)

    e0 = pl.multiple_of(wid * EPT, 8)
    pltpu.sync_copy(src_hbm.at[pl.ds(e0, EPT)], src_vm)
    pltpu.sync_copy(dst_hbm.at[wid], dst_vm)

    for p, sh_hbm in ((0, shl_hbm), (1, shr_hbm)):
        pltpu.sync_copy(sh_hbm.at[pl.ds(r0, RPT)], acc.at[pl.ds(r0, RPT)])

        @pl.when(sid == 0)
        def _():
            pltpu.sync_copy(sh_hbm.at[pl.ds(NS * RPT, RREM)],
                            acc.at[pl.ds(NS * RPT, RREM)])

        plsc.subcore_barrier()

        def _gather(k, b):
            off = pl.multiple_of(k * CH, 8)
            pltpu.make_async_copy(sh_hbm.at[src_vm.at[pl.ds(off, CH)]],
                                  rows_v.at[b], gsem.at[b]).start()

        def _wait_gather(b):
            pltpu.make_async_copy(sh_hbm.at[src_vm.at[pl.ds(0, CH)]],
                                  rows_v.at[b], gsem.at[b]).wait()

        def _scatter(k, b):
            pltpu.make_async_copy(rows_v.at[b], acc.at[dst_vm.at[k]],
                                  ssem.at[b]).start(add=True)

        def _wait_scatter(b):
            pltpu.make_async_copy(rows_v.at[b], acc.at[dst_vm.at[0]],
                                  ssem.at[b]).wait()

        for b in range(NBUF - 1):
            _gather(b, b)

        def outer(t, carry):
            k0 = NBUF * t
            for b in range(NBUF):
                k = k0 + b
                _wait_gather(b)
                _scatter(k, b)
                bn = (b + NBUF - 1) % NBUF
                if b == 0:
                    @pl.when(t >= 1)
                    def _():
                        _wait_scatter(bn)
                        _gather(k + NBUF - 1, bn)

                    @pl.when(t < 1)
                    def _():
                        _gather(k + NBUF - 1, bn)
                else:
                    @pl.when(k + NBUF - 1 < CPT)
                    def _():
                        _wait_scatter(bn)
                        _gather(k + NBUF - 1, bn)
            return carry

        lax.fori_loop(0, CPT // NBUF, outer, 0)
        for b in range(NBUF):
            _wait_scatter(b)
        plsc.subcore_barrier()
        pltpu.sync_copy(acc.at[pl.ds(r0, RPT)],
                        out_hbm.at[cid].at[p].at[pl.ds(r0, RPT)])

        @pl.when(sid == 0)
        def _():
            pltpu.sync_copy(acc.at[pl.ds(NS * RPT, RREM)],
                            out_hbm.at[cid].at[p].at[pl.ds(NS * RPT, RREM)])


_agg_kernel = functools.partial(
    pl.kernel,
    out_type=jax.ShapeDtypeStruct((NC, 2, N, DH), jnp.float32),
    mesh=_sc_mesh,
    compiler_params=pltpu.CompilerParams(use_tc_tiling_on_sc=False),
    scratch_types=[
        pltpu.VMEM((EPT,), jnp.int32),
        pltpu.VMEM((CPT, CH), jnp.int32),
        pltpu.VMEM((NBUF, CH, DH), jnp.float32),
        pltpu.SemaphoreType.DMA((NBUF,)),
        pltpu.SemaphoreType.DMA((NBUF,)),
        pltpu.VMEM_SHARED((N + TRASH, DH), jnp.float32),
    ],
)(_agg_body)


_RB = 1000


def _mm_body(x_ref, w_ref, hist_ref, shl_ref, shr_ref):
    deg = hist_ref[0, :, 0] + hist_ref[1, :, 0] - 1.0
    u = lax.rsqrt(deg)
    h = jnp.dot(x_ref[...], w_ref[...], preferred_element_type=jnp.float32)
    sh = h * u[:, None]
    shl_ref[...] = sh[:, :DH]
    shr_ref[...] = sh[:, DH:]


def _mm_call(x, w, hist):
    return pl.pallas_call(
        _mm_body,
        grid=(N // _RB,),
        in_specs=[
            pl.BlockSpec((_RB, D), lambda i: (i, 0)),
            pl.BlockSpec((D, D), lambda i: (0, 0)),
            pl.BlockSpec((NC, _RB, 16), lambda i: (0, i, 0)),
        ],
        out_specs=[
            pl.BlockSpec((_RB, DH), lambda i: (i, 0)),
            pl.BlockSpec((_RB, DH), lambda i: (i, 0)),
        ],
        out_shape=[
            jax.ShapeDtypeStruct((N, DH), jnp.float32),
            jax.ShapeDtypeStruct((N, DH), jnp.float32),
        ],
    )(x, w, hist)


def _fin_body(part_ref, shl_ref, shr_ref, hist_ref, b_ref, o_ref):
    deg = hist_ref[0, :, 0] + hist_ref[1, :, 0] - 1.0
    u = lax.rsqrt(deg)
    aggl = part_ref[0, 0] + part_ref[1, 0] - shl_ref[...]
    aggr = part_ref[0, 1] + part_ref[1, 1] - shr_ref[...]
    agg = jnp.concatenate([aggl, aggr], axis=-1)
    o_ref[...] = jnp.maximum(agg * u[:, None] + b_ref[...][None, :], 0.0)


def _fin_call(part, shl, shr, hist, b):
    return pl.pallas_call(
        _fin_body,
        grid=(N // _RB,),
        in_specs=[
            pl.BlockSpec((NC, 2, _RB, DH), lambda i: (0, 0, i, 0)),
            pl.BlockSpec((_RB, DH), lambda i: (i, 0)),
            pl.BlockSpec((_RB, DH), lambda i: (i, 0)),
            pl.BlockSpec((NC, _RB, 16), lambda i: (0, i, 0)),
            pl.BlockSpec((D,), lambda i: (0,)),
        ],
        out_specs=pl.BlockSpec((_RB, D), lambda i: (i, 0)),
        out_shape=jax.ShapeDtypeStruct((N, D), jnp.float32),
    )(part, shl, shr, hist, b)


def kernel(x, edge_index, W, b):
    npad = EPAD - E
    spread = jnp.arange(npad, dtype=jnp.int32) % N
    src = jnp.concatenate([edge_index[0], spread])
    trash = N + (jnp.arange(npad, dtype=jnp.int32) % TRASH)
    dst = jnp.concatenate([edge_index[1], trash]).reshape(NW, CPT, CH)
    ones = jnp.ones((N, 16), dtype=jnp.float32)
    hist = _deg_kernel(dst, ones)
    shl, shr = _mm_call(x, W, hist)
    part = _agg_kernel(shl, shr, src, dst)
    return _fin_call(part, shl, shr, hist, b)


# speedup vs baseline: 3.5575x; 1.0147x over previous
import functools

import jax
import jax.numpy as jnp
from jax import lax
from jax.experimental import pallas as pl
from jax.experimental.pallas import tpu as pltpu
import jax.experimental.pallas.tpu_sc as plsc

N = 10000
E = 320000
D = 128
NC = 2
NS = 16
NW = NC * NS
CH = 80
NBUF = 5
CPT = 125
EPT = CPT * CH
HW = 8
RPT = 624
RREM = N - NS * RPT
WIN = 8

_sc_mesh = plsc.VectorSubcoreMesh(core_axis_name="c", subcore_axis_name="s")


def _deg_body(dst_hbm, ones_hbm, hist_hbm, dst_vm, ones_v, ssem, acc):
    cid = lax.axis_index("c")
    sid = lax.axis_index("s")
    wid = sid * NC + cid
    r0 = pl.multiple_of(sid * RPT, 8)
    pltpu.sync_copy(ones_hbm.at[pl.ds(r0, RPT)], acc.at[pl.ds(r0, RPT)])

    @pl.when(sid == 0)
    def _():
        pltpu.sync_copy(ones_hbm.at[pl.ds(NS * RPT, RREM)],
                        acc.at[pl.ds(NS * RPT, RREM)])

    pltpu.sync_copy(dst_hbm.at[wid], dst_vm)
    pltpu.sync_copy(ones_hbm.at[pl.ds(0, CH)], ones_v)
    plsc.subcore_barrier()

    def body(k, carry):
        @pl.when(k >= WIN)
        def _():
            pltpu.make_async_copy(ones_v, acc.at[dst_vm.at[0]], ssem).wait()

        pltpu.make_async_copy(ones_v, acc.at[dst_vm.at[k]], ssem).start(add=True)
        return carry

    lax.fori_loop(0, CPT, body, 0)
    for _ in range(WIN):
        pltpu.make_async_copy(ones_v, acc.at[dst_vm.at[0]], ssem).wait()
    plsc.subcore_barrier()
    pltpu.sync_copy(acc.at[pl.ds(r0, RPT)], hist_hbm.at[cid].at[pl.ds(r0, RPT)])

    @pl.when(sid == 0)
    def _():
        pltpu.sync_copy(acc.at[pl.ds(NS * RPT, RREM)],
                        hist_hbm.at[cid].at[pl.ds(NS * RPT, RREM)])


_deg_kernel = functools.partial(
    pl.kernel,
    out_type=jax.ShapeDtypeStruct((NC, N, HW), jnp.float32),
    mesh=_sc_mesh,
    scratch_types=[
        pltpu.VMEM((CPT, CH), jnp.int32),
        pltpu.VMEM((CH, HW), jnp.float32),
        pltpu.SemaphoreType.DMA,
        pltpu.VMEM_SHARED((N, HW), jnp.float32),
    ],
)(_deg_body)


DH = D // 2


def _agg_body(shl_hbm, shr_hbm, src_hbm, dst_hbm, out_hbm,
              src_vm, dst_vm, rows_v, gsem, ssem, acc):
    cid = lax.axis_index("c")
    sid = lax.axis_index("s")
    wid = sid * NC + cid
    r0 = pl.multiple_of(sid * RPT, 8)

    e0 = pl.multiple_of(wid * EPT, 8)
    pltpu.sync_copy(src_hbm.at[pl.ds(e0, EPT)], src_vm)
    pltpu.sync_copy(dst_hbm.at[wid], dst_vm)

    for p, sh_hbm in ((0, shl_hbm), (1, shr_hbm)):
        pltpu.sync_copy(sh_hbm.at[pl.ds(r0, RPT)], acc.at[pl.ds(r0, RPT)])

        @pl.when(sid == 0)
        def _():
            pltpu.sync_copy(sh_hbm.at[pl.ds(NS * RPT, RREM)],
                            acc.at[pl.ds(NS * RPT, RREM)])

        plsc.subcore_barrier()

        def _gather(k, b):
            off = pl.multiple_of(k * CH, 8)
            pltpu.make_async_copy(sh_hbm.at[src_vm.at[pl.ds(off, CH)]],
                                  rows_v.at[b], gsem.at[b]).start()

        def _wait_gather(b):
            pltpu.make_async_copy(sh_hbm.at[src_vm.at[pl.ds(0, CH)]],
                                  rows_v.at[b], gsem.at[b]).wait()

        def _scatter(k, b):
            pltpu.make_async_copy(rows_v.at[b], acc.at[dst_vm.at[k]],
                                  ssem.at[b]).start(add=True)

        def _wait_scatter(b):
            pltpu.make_async_copy(rows_v.at[b], acc.at[dst_vm.at[0]],
                                  ssem.at[b]).wait()

        for b in range(NBUF - 1):
            _gather(b, b)

        def outer(t, carry):
            k0 = NBUF * t
            for b in range(NBUF):
                k = k0 + b
                _wait_gather(b)
                _scatter(k, b)
                bn = (b + NBUF - 1) % NBUF
                if b == 0:
                    @pl.when(t >= 1)
                    def _():
                        _wait_scatter(bn)
                        _gather(k + NBUF - 1, bn)

                    @pl.when(t < 1)
                    def _():
                        _gather(k + NBUF - 1, bn)
                else:
                    @pl.when(k + NBUF - 1 < CPT)
                    def _():
                        _wait_scatter(bn)
                        _gather(k + NBUF - 1, bn)
            return carry

        lax.fori_loop(0, CPT // NBUF, outer, 0)
        for b in range(NBUF):
            _wait_scatter(b)
        plsc.subcore_barrier()
        pltpu.sync_copy(acc.at[pl.ds(r0, RPT)],
                        out_hbm.at[cid].at[p].at[pl.ds(r0, RPT)])

        @pl.when(sid == 0)
        def _():
            pltpu.sync_copy(acc.at[pl.ds(NS * RPT, RREM)],
                            out_hbm.at[cid].at[p].at[pl.ds(NS * RPT, RREM)])


_agg_kernel = functools.partial(
    pl.kernel,
    out_type=jax.ShapeDtypeStruct((NC, 2, N, DH), jnp.float32),
    mesh=_sc_mesh,
    compiler_params=pltpu.CompilerParams(use_tc_tiling_on_sc=False),
    scratch_types=[
        pltpu.VMEM((EPT,), jnp.int32),
        pltpu.VMEM((CPT, CH), jnp.int32),
        pltpu.VMEM((NBUF, CH, DH), jnp.float32),
        pltpu.SemaphoreType.DMA((NBUF,)),
        pltpu.SemaphoreType.DMA((NBUF,)),
        pltpu.VMEM_SHARED((N, DH), jnp.float32),
    ],
)(_agg_body)


_RB = 1000


def _mm_body(x_ref, w_ref, hist_ref, shl_ref, shr_ref):
    deg = hist_ref[0, :, 0] + hist_ref[1, :, 0] - 1.0
    u = lax.rsqrt(deg)
    h = jnp.dot(x_ref[...], w_ref[...], preferred_element_type=jnp.float32)
    sh = h * u[:, None]
    shl_ref[...] = sh[:, :DH]
    shr_ref[...] = sh[:, DH:]


def _mm_call(x, w, hist):
    return pl.pallas_call(
        _mm_body,
        grid=(N // _RB,),
        in_specs=[
            pl.BlockSpec((_RB, D), lambda i: (i, 0)),
            pl.BlockSpec((D, D), lambda i: (0, 0)),
            pl.BlockSpec((NC, _RB, HW), lambda i: (0, i, 0)),
        ],
        out_specs=[
            pl.BlockSpec((_RB, DH), lambda i: (i, 0)),
            pl.BlockSpec((_RB, DH), lambda i: (i, 0)),
        ],
        out_shape=[
            jax.ShapeDtypeStruct((N, DH), jnp.float32),
            jax.ShapeDtypeStruct((N, DH), jnp.float32),
        ],
    )(x, w, hist)


def _fin_body(part_ref, shl_ref, shr_ref, hist_ref, b_ref, o_ref):
    deg = hist_ref[0, :, 0] + hist_ref[1, :, 0] - 1.0
    u = lax.rsqrt(deg)
    aggl = part_ref[0, 0] + part_ref[1, 0] - shl_ref[...]
    aggr = part_ref[0, 1] + part_ref[1, 1] - shr_ref[...]
    agg = jnp.concatenate([aggl, aggr], axis=-1)
    o_ref[...] = jnp.maximum(agg * u[:, None] + b_ref[...][None, :], 0.0)


def _fin_call(part, shl, shr, hist, b):
    return pl.pallas_call(
        _fin_body,
        grid=(N // _RB,),
        in_specs=[
            pl.BlockSpec((NC, 2, _RB, DH), lambda i: (0, 0, i, 0)),
            pl.BlockSpec((_RB, DH), lambda i: (i, 0)),
            pl.BlockSpec((_RB, DH), lambda i: (i, 0)),
            pl.BlockSpec((NC, _RB, HW), lambda i: (0, i, 0)),
            pl.BlockSpec((D,), lambda i: (0,)),
        ],
        out_specs=pl.BlockSpec((_RB, D), lambda i: (i, 0)),
        out_shape=jax.ShapeDtypeStruct((N, D), jnp.float32),
    )(part, shl, shr, hist, b)


def kernel(x, edge_index, W, b):
    src = edge_index[0]
    dst = edge_index[1].reshape(NW, CPT, CH)
    ones = jnp.ones((N, HW), dtype=jnp.float32)
    hist = _deg_kernel(dst, ones)
    shl, shr = _mm_call(x, W, hist)
    part = _agg_kernel(shl, shr, src, dst)
    return _fin_call(part, shl, shr, hist, b)
